# Initial kernel scaffold; baseline (speedup 1.0000x reference)
import jax
import jax.numpy as jnp
from jax.experimental import pallas as pl


def kernel(h, r, t, tt, c_h, c_r, c_t, c_tt, a_embedding, w_embedding, b_embedding, relation_embedding):
    raise NotImplementedError("write your pallas kernel here")


# speedup vs baseline: 1.2344x; 1.2344x over previous
import functools
import math

import jax
import jax.numpy as jnp
from jax import lax
from jax.experimental import pallas as pl
from jax.experimental.pallas import tpu as pltpu
from jax.experimental.pallas import tpu_sc as plsc

ENTITY_NUM = 100000
RELATION_NUM = 500
EMBED_DIM = 64
DMOD = 19
BATCH = 16384
NUM_CORES = 2
NUM_SUBCORES = 16
NUM_WORKERS = NUM_CORES * NUM_SUBCORES
PER_W = BATCH // NUM_WORKERS
CHUNK = 128
NCHUNK = PER_W // CHUNK
GROUPS = CHUNK // 16

_INV_PI = float(1.0 / math.pi)
_PI_HI = 3.140625
_PI_LO = float(math.pi - 3.140625)
_MAGIC = 12582912.0
_S3 = -1.6666654611e-1
_S5 = 8.3321608736e-3
_S7 = -1.9515295891e-4


def _sin16(x):
    kf = x * _INV_PI
    kf = (kf + _MAGIC) - _MAGIC
    r = x - kf * _PI_HI
    r = r - kf * _PI_LO
    k = kf.astype(jnp.int32)
    r2 = r * r
    p = _S5 + r2 * _S7
    p = _S3 + r2 * p
    s = r + r * (r2 * p)
    return jnp.where((k & 1) == 1, -s, s)


def _body(h_hbm, r_hbm, t_hbm, tt_hbm, c_h_hbm, c_r_hbm, c_t_hbm, c_tt_hbm,
          a_hbm, w_hbm, b_hbm, rel_hbm, correct_hbm, corrupt_hbm,
          e1_v, e2_v, ri_v, tt_v, a1_v, w1_v, b1_v, a2_v, w2_v, b2_v,
          rel_v, out_v, sem):
    wid = lax.axis_index("s") * NUM_CORES + lax.axis_index("c")
    wbase = wid * PER_W

    pltpu.sync_copy(rel_hbm, rel_v)

    def process(e1_hbm, rr_hbm, e2_hbm, time_hbm, out_hbm):
        for ci in range(NCHUNK):
            base = wbase + ci * CHUNK
            sl = pl.ds(base, CHUNK)
            pltpu.sync_copy(e1_hbm.at[sl], e1_v)
            pltpu.sync_copy(e2_hbm.at[sl], e2_v)
            pltpu.sync_copy(rr_hbm.at[sl], ri_v)
            pltpu.sync_copy(time_hbm.at[sl], tt_v)
            cps = [
                pltpu.async_copy(a_hbm.at[e1_v], a1_v, sem),
                pltpu.async_copy(w_hbm.at[e1_v], w1_v, sem),
                pltpu.async_copy(b_hbm.at[e1_v], b1_v, sem),
                pltpu.async_copy(a_hbm.at[e2_v], a2_v, sem),
                pltpu.async_copy(w_hbm.at[e2_v], w2_v, sem),
                pltpu.async_copy(b_hbm.at[e2_v], b2_v, sem),
            ]
            for cp in cps:
                cp.wait()

            @pl.loop(0, GROUPS)
            def _group(g):
                row0 = g * 16
                rows = lax.iota(jnp.int32, 16) + row0
                ttf = tt_v[pl.ds(row0, 16)].astype(jnp.float32)
                r16 = ri_v[pl.ds(row0, 16)]
                acc = jnp.zeros((16,), jnp.float32)
                for j in range(EMBED_DIM):
                    colj = jnp.full((16,), j, jnp.int32)
                    a1 = plsc.load_gather(a1_v, [rows, colj])
                    a2 = plsc.load_gather(a2_v, [rows, colj])
                    re = plsc.load_gather(rel_v, [r16, colj])
                    if j < DMOD:
                        w1 = plsc.load_gather(w1_v, [rows, colj])
                        b1 = plsc.load_gather(b1_v, [rows, colj])
                        w2 = plsc.load_gather(w2_v, [rows, colj])
                        b2 = plsc.load_gather(b2_v, [rows, colj])
                        d = (a1 * _sin16(w1 * ttf + b1) + re
                             - a2 * _sin16(w2 * ttf + b2))
                    else:
                        d = a1 + re - a2
                    acc += d * d
                out_v[pl.ds(row0, 16)] = acc

            pltpu.sync_copy(out_v, out_hbm.at[sl])

    process(h_hbm, r_hbm, t_hbm, tt_hbm, correct_hbm)
    process(c_h_hbm, c_r_hbm, c_t_hbm, c_tt_hbm, corrupt_hbm)


@jax.jit
def _run(h, r, t, tt, c_h, c_r, c_t, c_tt, a_emb, w_emb, b_emb, rel_emb):
    f = pl.kernel(
        _body,
        out_type=(
            jax.ShapeDtypeStruct((BATCH,), jnp.float32),
            jax.ShapeDtypeStruct((BATCH,), jnp.float32),
        ),
        compiler_params=pltpu.CompilerParams(
            needs_layout_passes=False, use_tc_tiling_on_sc=False),
        mesh=plsc.VectorSubcoreMesh(
            core_axis_name="c", subcore_axis_name="s",
            num_cores=NUM_CORES, num_subcores=NUM_SUBCORES),
        scratch_types=[
            pltpu.VMEM((CHUNK,), jnp.int32),
            pltpu.VMEM((CHUNK,), jnp.int32),
            pltpu.VMEM((CHUNK,), jnp.int32),
            pltpu.VMEM((CHUNK,), jnp.int32),
            pltpu.VMEM((CHUNK, EMBED_DIM), jnp.float32),
            pltpu.VMEM((CHUNK, DMOD), jnp.float32),
            pltpu.VMEM((CHUNK, DMOD), jnp.float32),
            pltpu.VMEM((CHUNK, EMBED_DIM), jnp.float32),
            pltpu.VMEM((CHUNK, DMOD), jnp.float32),
            pltpu.VMEM((CHUNK, DMOD), jnp.float32),
            pltpu.VMEM((RELATION_NUM, EMBED_DIM), jnp.float32),
            pltpu.VMEM((CHUNK,), jnp.float32),
            pltpu.SemaphoreType.DMA,
        ],
    )
    return f(h, r, t, tt, c_h, c_r, c_t, c_tt, a_emb, w_emb, b_emb, rel_emb)


def kernel(h, r, t, tt, c_h, c_r, c_t, c_tt,
           a_embedding, w_embedding, b_embedding, relation_embedding):
    i32 = jnp.int32
    out = _run(h.astype(i32), r.astype(i32), t.astype(i32), tt.astype(i32),
               c_h.astype(i32), c_r.astype(i32), c_t.astype(i32),
               c_tt.astype(i32), a_embedding, w_embedding, b_embedding,
               relation_embedding)
    return (out[0], out[1])


# speedup vs baseline: 1.7814x; 1.4431x over previous
import functools
import math

import jax
import jax.numpy as jnp
from jax import lax
from jax.experimental import pallas as pl
from jax.experimental.pallas import tpu as pltpu
from jax.experimental.pallas import tpu_sc as plsc

ENTITY_NUM = 100000
HALF_N = ENTITY_NUM // 2
RELATION_NUM = 500
EMBED_DIM = 64
DMOD = 19
BOFF = 24
BATCH = 16384
NUM_CORES = 2
NUM_SUBCORES = 16
NUM_WORKERS = NUM_CORES * NUM_SUBCORES
PER_W = BATCH // NUM_WORKERS
CHUNK = 128
NCHUNK = PER_W // CHUNK
GROUPS = CHUNK // 16
ROWW = 128
TB = 2048
NTB = (ENTITY_NUM + TB - 1) // TB

_INV_PI = float(1.0 / math.pi)
_PI_HI = 3.140625
_PI_LO = float(math.pi - 3.140625)
_MAGIC = 12582912.0
_S3 = -1.6666654611e-1
_S5 = 8.3321608736e-3
_S7 = -1.9515295891e-4


def _sin16(x):
    kf = x * _INV_PI
    kf = (kf + _MAGIC) - _MAGIC
    r = x - kf * _PI_HI
    r = r - kf * _PI_LO
    k = kf.astype(jnp.int32)
    r2 = r * r
    p = _S5 + r2 * _S7
    p = _S3 + r2 * p
    s = r + r * (r2 * p)
    return jnp.where((k & 1) == 1, -s, s)


def _repack_body(a_ref, w_ref, b_ref, ao_ref, wbo_ref):
    ao_ref[...] = jnp.concatenate(
        [a_ref[...].T, jnp.zeros((TB, ROWW - EMBED_DIM), jnp.float32)],
        axis=1)
    z5 = jnp.zeros((TB, BOFF - DMOD), jnp.float32)
    ztail = jnp.zeros((TB, ROWW - BOFF - DMOD), jnp.float32)
    wbo_ref[...] = jnp.concatenate(
        [w_ref[...].T, z5, b_ref[...].T, ztail], axis=1)


def _repack(at, wt, bt):
    return pl.pallas_call(
        _repack_body,
        grid=(NTB,),
        in_specs=[
            pl.BlockSpec((EMBED_DIM, TB), lambda i: (0, i)),
            pl.BlockSpec((DMOD, TB), lambda i: (0, i)),
            pl.BlockSpec((DMOD, TB), lambda i: (0, i)),
        ],
        out_specs=[
            pl.BlockSpec((TB, ROWW), lambda i: (i, 0)),
            pl.BlockSpec((TB, ROWW), lambda i: (i, 0)),
        ],
        out_shape=[
            jax.ShapeDtypeStruct((ENTITY_NUM, ROWW), jnp.float32),
            jax.ShapeDtypeStruct((ENTITY_NUM, ROWW), jnp.float32),
        ],
    )(at, wt, bt)


def _body(h_hbm, r_hbm, t_hbm, tt_hbm, c_h_hbm, c_r_hbm, c_t_hbm, c_tt_hbm,
          a_hbm, wb_hbm, rel_hbm, correct_hbm, corrupt_hbm,
          hi_v, ti_v, ri_v, tti_v, chi_v, cti_v, cri_v, ctti_v,
          a1_v, wb1_v, a2_v, wb2_v, rel_v, out_v, sem):
    wid = lax.axis_index("s") * NUM_CORES + lax.axis_index("c")
    wbase = wid * PER_W

    pltpu.sync_copy(h_hbm.at[pl.ds(wbase, PER_W)], hi_v)
    pltpu.sync_copy(t_hbm.at[pl.ds(wbase, PER_W)], ti_v)
    pltpu.sync_copy(r_hbm.at[pl.ds(wbase, PER_W)], ri_v)
    pltpu.sync_copy(tt_hbm.at[pl.ds(wbase, PER_W)], tti_v)
    pltpu.sync_copy(c_h_hbm.at[pl.ds(wbase, PER_W)], chi_v)
    pltpu.sync_copy(c_t_hbm.at[pl.ds(wbase, PER_W)], cti_v)
    pltpu.sync_copy(c_r_hbm.at[pl.ds(wbase, PER_W)], cri_v)
    pltpu.sync_copy(c_tt_hbm.at[pl.ds(wbase, PER_W)], ctti_v)
    pltpu.sync_copy(rel_hbm, rel_v)

    def process(e1_v, e2_v, rr_v, time_v, out_hbm):
        @pl.loop(0, NCHUNK)
        def _chunk(ci):
            cs = pl.ds(ci * CHUNK, CHUNK)
            cps = [
                pltpu.async_copy(a_hbm.at[e1_v.at[cs]], a1_v, sem),
                pltpu.async_copy(wb_hbm.at[e1_v.at[cs]], wb1_v, sem),
                pltpu.async_copy(a_hbm.at[e2_v.at[cs]], a2_v, sem),
                pltpu.async_copy(wb_hbm.at[e2_v.at[cs]], wb2_v, sem),
            ]
            for cp in cps:
                cp.wait()

            @pl.loop(0, GROUPS)
            def _group(g):
                row0 = g * 16
                grow0 = ci * CHUNK + row0
                rows = lax.iota(jnp.int32, 16) + row0
                ttf = time_v[pl.ds(grow0, 16)].astype(jnp.float32)
                r16 = rr_v[pl.ds(grow0, 16)]
                acc = jnp.zeros((16,), jnp.float32)
                for j in range(EMBED_DIM):
                    colj = jnp.full((16,), j, jnp.int32)
                    a1 = plsc.load_gather(a1_v, [rows, colj])
                    a2 = plsc.load_gather(a2_v, [rows, colj])
                    re = plsc.load_gather(rel_v, [r16, colj])
                    if j < DMOD:
                        colb = jnp.full((16,), BOFF + j, jnp.int32)
                        w1 = plsc.load_gather(wb1_v, [rows, colj])
                        b1 = plsc.load_gather(wb1_v, [rows, colb])
                        w2 = plsc.load_gather(wb2_v, [rows, colj])
                        b2 = plsc.load_gather(wb2_v, [rows, colb])
                        d = (a1 * _sin16(w1 * ttf + b1) + re
                             - a2 * _sin16(w2 * ttf + b2))
                    else:
                        d = a1 + re - a2
                    acc += d * d
                out_v[pl.ds(grow0, 16)] = acc

        pltpu.sync_copy(out_v, out_hbm.at[pl.ds(wbase, PER_W)])

    process(hi_v, ti_v, ri_v, tti_v, correct_hbm)
    process(chi_v, cti_v, cri_v, ctti_v, corrupt_hbm)


def _score(h, r, t, tt, c_h, c_r, c_t, c_tt, a_rows, wb_rows, rel_emb):
    f = pl.kernel(
        _body,
        out_type=(
            jax.ShapeDtypeStruct((BATCH,), jnp.float32),
            jax.ShapeDtypeStruct((BATCH,), jnp.float32),
        ),
        compiler_params=pltpu.CompilerParams(
            needs_layout_passes=False, use_tc_tiling_on_sc=False),
        mesh=plsc.VectorSubcoreMesh(
            core_axis_name="c", subcore_axis_name="s",
            num_cores=NUM_CORES, num_subcores=NUM_SUBCORES),
        scratch_types=[
            pltpu.VMEM((PER_W,), jnp.int32),
            pltpu.VMEM((PER_W,), jnp.int32),
            pltpu.VMEM((PER_W,), jnp.int32),
            pltpu.VMEM((PER_W,), jnp.int32),
            pltpu.VMEM((PER_W,), jnp.int32),
            pltpu.VMEM((PER_W,), jnp.int32),
            pltpu.VMEM((PER_W,), jnp.int32),
            pltpu.VMEM((PER_W,), jnp.int32),
            pltpu.VMEM((CHUNK, ROWW), jnp.float32),
            pltpu.VMEM((CHUNK, ROWW), jnp.float32),
            pltpu.VMEM((CHUNK, ROWW), jnp.float32),
            pltpu.VMEM((CHUNK, ROWW), jnp.float32),
            pltpu.VMEM((RELATION_NUM, EMBED_DIM), jnp.float32),
            pltpu.VMEM((PER_W,), jnp.float32),
            pltpu.SemaphoreType.DMA,
        ],
    )
    return f(h, r, t, tt, c_h, c_r, c_t, c_tt, a_rows, wb_rows, rel_emb)


def kernel(h, r, t, tt, c_h, c_r, c_t, c_tt,
           a_embedding, w_embedding, b_embedding, relation_embedding):
    i32 = jnp.int32
    a_pack, wb_pack = _repack(jnp.transpose(a_embedding),
                              jnp.transpose(w_embedding),
                              jnp.transpose(b_embedding))
    a_rows = a_pack
    wb_rows = wb_pack
    out = _score(h.astype(i32), r.astype(i32), t.astype(i32), tt.astype(i32),
                 c_h.astype(i32), c_r.astype(i32), c_t.astype(i32),
                 c_tt.astype(i32), a_rows, wb_rows, relation_embedding)
    return (out[0], out[1])


# speedup vs baseline: 1.9033x; 1.0684x over previous
import functools
import math

import jax
import jax.numpy as jnp
from jax import lax
from jax.experimental import pallas as pl
from jax.experimental.pallas import tpu as pltpu
from jax.experimental.pallas import tpu_sc as plsc

ENTITY_NUM = 100000
HALF_N = ENTITY_NUM // 2
RELATION_NUM = 500
EMBED_DIM = 64
DMOD = 19
BOFF = 24
BATCH = 16384
NUM_CORES = 2
NUM_SUBCORES = 16
NUM_WORKERS = NUM_CORES * NUM_SUBCORES
PER_W = BATCH // NUM_WORKERS
CHUNK = 64
NCHUNK = PER_W // CHUNK
GROUPS = CHUNK // 16
ROWW = 128
TB = 2048
NTB = (ENTITY_NUM + TB - 1) // TB

_INV_PI = float(1.0 / math.pi)
_PI_HI = 3.140625
_PI_LO = float(math.pi - 3.140625)
_MAGIC = 12582912.0
_S3 = -1.6666654611e-1
_S5 = 8.3321608736e-3
_S7 = -1.9515295891e-4


def _sin16(x):
    kf = x * _INV_PI
    kf = (kf + _MAGIC) - _MAGIC
    r = x - kf * _PI_HI
    r = r - kf * _PI_LO
    k = kf.astype(jnp.int32)
    r2 = r * r
    p = _S5 + r2 * _S7
    p = _S3 + r2 * p
    s = r + r * (r2 * p)
    return jnp.where((k & 1) == 1, -s, s)


def _repack_body(a_ref, w_ref, b_ref, ao_ref, wbo_ref):
    ao_ref[...] = jnp.concatenate(
        [a_ref[...].T, jnp.zeros((TB, ROWW - EMBED_DIM), jnp.float32)],
        axis=1)
    z5 = jnp.zeros((TB, BOFF - DMOD), jnp.float32)
    ztail = jnp.zeros((TB, ROWW - BOFF - DMOD), jnp.float32)
    wbo_ref[...] = jnp.concatenate(
        [w_ref[...].T, z5, b_ref[...].T, ztail], axis=1)


def _repack(at, wt, bt):
    return pl.pallas_call(
        _repack_body,
        grid=(NTB,),
        in_specs=[
            pl.BlockSpec((EMBED_DIM, TB), lambda i: (0, i)),
            pl.BlockSpec((DMOD, TB), lambda i: (0, i)),
            pl.BlockSpec((DMOD, TB), lambda i: (0, i)),
        ],
        out_specs=[
            pl.BlockSpec((TB, ROWW), lambda i: (i, 0)),
            pl.BlockSpec((TB, ROWW), lambda i: (i, 0)),
        ],
        out_shape=[
            jax.ShapeDtypeStruct((ENTITY_NUM, ROWW), jnp.float32),
            jax.ShapeDtypeStruct((ENTITY_NUM, ROWW), jnp.float32),
        ],
    )(at, wt, bt)


def _body(h_hbm, r_hbm, t_hbm, tt_hbm, c_h_hbm, c_r_hbm, c_t_hbm, c_tt_hbm,
          a_hbm, wb_hbm, rel_hbm, correct_hbm, corrupt_hbm,
          hi_v, ti_v, ri_v, tti_v, chi_v, cti_v, cri_v, ctti_v,
          a1_v, wb1_v, a2_v, wb2_v, a1b_v, wb1b_v, a2b_v, wb2b_v,
          rel_v, out_v, sem_a, sem_b):
    wid = lax.axis_index("s") * NUM_CORES + lax.axis_index("c")
    wbase = wid * PER_W

    pltpu.sync_copy(h_hbm.at[pl.ds(wbase, PER_W)], hi_v)
    pltpu.sync_copy(t_hbm.at[pl.ds(wbase, PER_W)], ti_v)
    pltpu.sync_copy(r_hbm.at[pl.ds(wbase, PER_W)], ri_v)
    pltpu.sync_copy(tt_hbm.at[pl.ds(wbase, PER_W)], tti_v)
    pltpu.sync_copy(c_h_hbm.at[pl.ds(wbase, PER_W)], chi_v)
    pltpu.sync_copy(c_t_hbm.at[pl.ds(wbase, PER_W)], cti_v)
    pltpu.sync_copy(c_r_hbm.at[pl.ds(wbase, PER_W)], cri_v)
    pltpu.sync_copy(c_tt_hbm.at[pl.ds(wbase, PER_W)], ctti_v)
    pltpu.sync_copy(rel_hbm, rel_v)

    bufs = ((a1_v, wb1_v, a2_v, wb2_v, sem_a),
            (a1b_v, wb1b_v, a2b_v, wb2b_v, sem_b))

    def process(e1_v, e2_v, rr_v, time_v, out_hbm):
        def fire(ci, par):
            A1, WB1, A2, WB2, sem = bufs[par]
            cs = pl.ds(ci * CHUNK, CHUNK)
            pltpu.async_copy(a_hbm.at[e1_v.at[cs]], A1, sem)
            pltpu.async_copy(wb_hbm.at[e1_v.at[cs]], WB1, sem)
            pltpu.async_copy(a_hbm.at[e2_v.at[cs]], A2, sem)
            pltpu.async_copy(wb_hbm.at[e2_v.at[cs]], WB2, sem)

        def drain(ci, par):
            A1, WB1, A2, WB2, sem = bufs[par]
            cs = pl.ds(ci * CHUNK, CHUNK)
            pltpu.make_async_copy(a_hbm.at[e1_v.at[cs]], A1, sem).wait()
            pltpu.make_async_copy(wb_hbm.at[e1_v.at[cs]], WB1, sem).wait()
            pltpu.make_async_copy(a_hbm.at[e2_v.at[cs]], A2, sem).wait()
            pltpu.make_async_copy(wb_hbm.at[e2_v.at[cs]], WB2, sem).wait()

        def compute(ci, par):
            A1, WB1, A2, WB2, _ = bufs[par]

            @pl.loop(0, GROUPS)
            def _group(g):
                row0 = g * 16
                grow0 = ci * CHUNK + row0
                rows = lax.iota(jnp.int32, 16) + row0
                ttf = time_v[pl.ds(grow0, 16)].astype(jnp.float32)
                r16 = rr_v[pl.ds(grow0, 16)]
                acc = jnp.zeros((16,), jnp.float32)
                for j in range(EMBED_DIM):
                    colj = jnp.full((16,), j, jnp.int32)
                    a1 = plsc.load_gather(A1, [rows, colj])
                    a2 = plsc.load_gather(A2, [rows, colj])
                    re = plsc.load_gather(rel_v, [r16, colj])
                    if j < DMOD:
                        colb = jnp.full((16,), BOFF + j, jnp.int32)
                        w1 = plsc.load_gather(WB1, [rows, colj])
                        b1 = plsc.load_gather(WB1, [rows, colb])
                        w2 = plsc.load_gather(WB2, [rows, colj])
                        b2 = plsc.load_gather(WB2, [rows, colb])
                        d = (a1 * _sin16(w1 * ttf + b1) + re
                             - a2 * _sin16(w2 * ttf + b2))
                    else:
                        d = a1 + re - a2
                    acc += d * d
                out_v[pl.ds(grow0, 16)] = acc

        fire(0, 0)

        @pl.loop(0, NCHUNK // 2)
        def _chunk2(cj):
            ci0 = cj * 2
            fire(ci0 + 1, 1)
            drain(ci0, 0)
            compute(ci0, 0)

            @pl.when(ci0 + 2 < NCHUNK)
            def _prefetch():
                fire(ci0 + 2, 0)

            drain(ci0 + 1, 1)
            compute(ci0 + 1, 1)

        pltpu.sync_copy(out_v, out_hbm.at[pl.ds(wbase, PER_W)])

    process(hi_v, ti_v, ri_v, tti_v, correct_hbm)
    process(chi_v, cti_v, cri_v, ctti_v, corrupt_hbm)


def _score(h, r, t, tt, c_h, c_r, c_t, c_tt, a_rows, wb_rows, rel_emb):
    f = pl.kernel(
        _body,
        out_type=(
            jax.ShapeDtypeStruct((BATCH,), jnp.float32),
            jax.ShapeDtypeStruct((BATCH,), jnp.float32),
        ),
        compiler_params=pltpu.CompilerParams(
            needs_layout_passes=False, use_tc_tiling_on_sc=False),
        mesh=plsc.VectorSubcoreMesh(
            core_axis_name="c", subcore_axis_name="s",
            num_cores=NUM_CORES, num_subcores=NUM_SUBCORES),
        scratch_types=[
            pltpu.VMEM((PER_W,), jnp.int32),
            pltpu.VMEM((PER_W,), jnp.int32),
            pltpu.VMEM((PER_W,), jnp.int32),
            pltpu.VMEM((PER_W,), jnp.int32),
            pltpu.VMEM((PER_W,), jnp.int32),
            pltpu.VMEM((PER_W,), jnp.int32),
            pltpu.VMEM((PER_W,), jnp.int32),
            pltpu.VMEM((PER_W,), jnp.int32),
            pltpu.VMEM((CHUNK, ROWW), jnp.float32),
            pltpu.VMEM((CHUNK, ROWW), jnp.float32),
            pltpu.VMEM((CHUNK, ROWW), jnp.float32),
            pltpu.VMEM((CHUNK, ROWW), jnp.float32),
            pltpu.VMEM((CHUNK, ROWW), jnp.float32),
            pltpu.VMEM((CHUNK, ROWW), jnp.float32),
            pltpu.VMEM((CHUNK, ROWW), jnp.float32),
            pltpu.VMEM((CHUNK, ROWW), jnp.float32),
            pltpu.VMEM((RELATION_NUM, EMBED_DIM), jnp.float32),
            pltpu.VMEM((PER_W,), jnp.float32),
            pltpu.SemaphoreType.DMA,
            pltpu.SemaphoreType.DMA,
        ],
    )
    return f(h, r, t, tt, c_h, c_r, c_t, c_tt, a_rows, wb_rows, rel_emb)


def kernel(h, r, t, tt, c_h, c_r, c_t, c_tt,
           a_embedding, w_embedding, b_embedding, relation_embedding):
    i32 = jnp.int32
    a_pack, wb_pack = _repack(jnp.transpose(a_embedding),
                              jnp.transpose(w_embedding),
                              jnp.transpose(b_embedding))
    a_rows = a_pack
    wb_rows = wb_pack
    out = _score(h.astype(i32), r.astype(i32), t.astype(i32), tt.astype(i32),
                 c_h.astype(i32), c_r.astype(i32), c_t.astype(i32),
                 c_tt.astype(i32), a_rows, wb_rows, relation_embedding)
    return (out[0], out[1])


# speedup vs baseline: 1.9053x; 1.0010x over previous
import functools
import math

import jax
import jax.numpy as jnp
from jax import lax
from jax.experimental import pallas as pl
from jax.experimental.pallas import tpu as pltpu
from jax.experimental.pallas import tpu_sc as plsc

ENTITY_NUM = 100000
HALF_N = ENTITY_NUM // 2
RELATION_NUM = 500
EMBED_DIM = 64
DMOD = 19
BOFF = 24
BATCH = 16384
NUM_CORES = 2
NUM_SUBCORES = 16
NUM_WORKERS = NUM_CORES * NUM_SUBCORES
PER_W = BATCH // NUM_WORKERS
CHUNK = 128
NCHUNK = PER_W // CHUNK
GROUPS = CHUNK // 16
ROWW = 128
TB = 2048
NTB = (ENTITY_NUM + TB - 1) // TB

_INV_PI = float(1.0 / math.pi)
_PI_HI = 3.140625
_PI_LO = float(math.pi - 3.140625)
_MAGIC = 12582912.0
_S3 = -1.6666654611e-1
_S5 = 8.3321608736e-3
_S7 = -1.9515295891e-4


def _sin16(x):
    kf = x * _INV_PI
    kf = (kf + _MAGIC) - _MAGIC
    r = x - kf * _PI_HI
    r = r - kf * _PI_LO
    k = kf.astype(jnp.int32)
    r2 = r * r
    p = _S5 + r2 * _S7
    p = _S3 + r2 * p
    s = r + r * (r2 * p)
    return jnp.where((k & 1) == 1, -s, s)


def _repack_body(a_ref, w_ref, b_ref, ao_ref, wbo_ref):
    ao_ref[...] = jnp.concatenate(
        [a_ref[...].T, jnp.zeros((TB, ROWW - EMBED_DIM), jnp.float32)],
        axis=1)
    z5 = jnp.zeros((TB, BOFF - DMOD), jnp.float32)
    ztail = jnp.zeros((TB, ROWW - BOFF - DMOD), jnp.float32)
    wbo_ref[...] = jnp.concatenate(
        [w_ref[...].T, z5, b_ref[...].T, ztail], axis=1)


def _repack(at, wt, bt):
    return pl.pallas_call(
        _repack_body,
        grid=(NTB,),
        in_specs=[
            pl.BlockSpec((EMBED_DIM, TB), lambda i: (0, i)),
            pl.BlockSpec((DMOD, TB), lambda i: (0, i)),
            pl.BlockSpec((DMOD, TB), lambda i: (0, i)),
        ],
        out_specs=[
            pl.BlockSpec((TB, ROWW), lambda i: (i, 0)),
            pl.BlockSpec((TB, ROWW), lambda i: (i, 0)),
        ],
        out_shape=[
            jax.ShapeDtypeStruct((ENTITY_NUM, ROWW), jnp.float32),
            jax.ShapeDtypeStruct((ENTITY_NUM, ROWW), jnp.float32),
        ],
    )(at, wt, bt)


def _body(h_hbm, r_hbm, t_hbm, tt_hbm, c_h_hbm, c_r_hbm, c_t_hbm, c_tt_hbm,
          a_hbm, wb_hbm, rel_hbm, correct_hbm, corrupt_hbm,
          hi_v, ti_v, ri_v, tti_v, chi_v, cti_v, cri_v, ctti_v,
          a1_v, wb1_v, a2_v, wb2_v, a1b_v, wb1b_v, a2b_v, wb2b_v,
          rel_v, out_v, sem_a, sem_b):
    wid = lax.axis_index("s") * NUM_CORES + lax.axis_index("c")
    wbase = wid * PER_W

    pltpu.sync_copy(h_hbm.at[pl.ds(wbase, PER_W)], hi_v)
    pltpu.sync_copy(t_hbm.at[pl.ds(wbase, PER_W)], ti_v)
    pltpu.sync_copy(r_hbm.at[pl.ds(wbase, PER_W)], ri_v)
    pltpu.sync_copy(tt_hbm.at[pl.ds(wbase, PER_W)], tti_v)
    pltpu.sync_copy(c_h_hbm.at[pl.ds(wbase, PER_W)], chi_v)
    pltpu.sync_copy(c_t_hbm.at[pl.ds(wbase, PER_W)], cti_v)
    pltpu.sync_copy(c_r_hbm.at[pl.ds(wbase, PER_W)], cri_v)
    pltpu.sync_copy(c_tt_hbm.at[pl.ds(wbase, PER_W)], ctti_v)
    pltpu.sync_copy(rel_hbm, rel_v)

    @pl.loop(0, PER_W // 16)
    def _remap(i):
        s = pl.ds(i * 16, 16)
        for ref in (hi_v, ti_v, chi_v, cti_v):
            ref[s] = ref[s] * 2

    bufs = ((a1_v, wb1_v, a2_v, wb2_v, sem_a),
            (a1b_v, wb1b_v, a2b_v, wb2b_v, sem_b))

    def process(e1_v, e2_v, rr_v, time_v, out_hbm):
        def fire(ci, par):
            A1, WB1, A2, WB2, sem = bufs[par]
            cs = pl.ds(ci * CHUNK, CHUNK)
            pltpu.async_copy(a_hbm.at[e1_v.at[cs]], A1, sem)
            pltpu.async_copy(wb_hbm.at[e1_v.at[cs]], WB1, sem)
            pltpu.async_copy(a_hbm.at[e2_v.at[cs]], A2, sem)
            pltpu.async_copy(wb_hbm.at[e2_v.at[cs]], WB2, sem)

        def drain(ci, par):
            A1, WB1, A2, WB2, sem = bufs[par]
            cs = pl.ds(ci * CHUNK, CHUNK)
            pltpu.make_async_copy(a_hbm.at[e1_v.at[cs]], A1, sem).wait()
            pltpu.make_async_copy(wb_hbm.at[e1_v.at[cs]], WB1, sem).wait()
            pltpu.make_async_copy(a_hbm.at[e2_v.at[cs]], A2, sem).wait()
            pltpu.make_async_copy(wb_hbm.at[e2_v.at[cs]], WB2, sem).wait()

        def compute(ci, par):
            A1, WB1, A2, WB2, _ = bufs[par]

            @pl.loop(0, GROUPS)
            def _group(g):
                row0 = g * 16
                grow0 = ci * CHUNK + row0
                rows = lax.iota(jnp.int32, 16) + row0
                ttf = time_v[pl.ds(grow0, 16)].astype(jnp.float32)
                r16 = rr_v[pl.ds(grow0, 16)]
                acc = jnp.zeros((16,), jnp.float32)
                for j in range(EMBED_DIM):
                    colj = jnp.full((16,), j, jnp.int32)
                    a1 = plsc.load_gather(A1, [rows, colj])
                    a2 = plsc.load_gather(A2, [rows, colj])
                    re = plsc.load_gather(rel_v, [r16, colj])
                    if j < DMOD:
                        colb = jnp.full((16,), BOFF + j, jnp.int32)
                        w1 = plsc.load_gather(WB1, [rows, colj])
                        b1 = plsc.load_gather(WB1, [rows, colb])
                        w2 = plsc.load_gather(WB2, [rows, colj])
                        b2 = plsc.load_gather(WB2, [rows, colb])
                        d = (a1 * _sin16(w1 * ttf + b1) + re
                             - a2 * _sin16(w2 * ttf + b2))
                    else:
                        d = a1 + re - a2
                    acc += d * d
                out_v[pl.ds(grow0, 16)] = acc

        fire(0, 0)

        @pl.loop(0, NCHUNK // 2)
        def _chunk2(cj):
            ci0 = cj * 2
            fire(ci0 + 1, 1)
            drain(ci0, 0)
            compute(ci0, 0)

            @pl.when(ci0 + 2 < NCHUNK)
            def _prefetch():
                fire(ci0 + 2, 0)

            drain(ci0 + 1, 1)
            compute(ci0 + 1, 1)

        pltpu.sync_copy(out_v, out_hbm.at[pl.ds(wbase, PER_W)])

    process(hi_v, ti_v, ri_v, tti_v, correct_hbm)
    process(chi_v, cti_v, cri_v, ctti_v, corrupt_hbm)


def _score(h, r, t, tt, c_h, c_r, c_t, c_tt, a_rows, wb_rows, rel_emb):
    f = pl.kernel(
        _body,
        out_type=(
            jax.ShapeDtypeStruct((BATCH,), jnp.float32),
            jax.ShapeDtypeStruct((BATCH,), jnp.float32),
        ),
        compiler_params=pltpu.CompilerParams(
            needs_layout_passes=False, use_tc_tiling_on_sc=False),
        mesh=plsc.VectorSubcoreMesh(
            core_axis_name="c", subcore_axis_name="s",
            num_cores=NUM_CORES, num_subcores=NUM_SUBCORES),
        scratch_types=[
            pltpu.VMEM((PER_W,), jnp.int32),
            pltpu.VMEM((PER_W,), jnp.int32),
            pltpu.VMEM((PER_W,), jnp.int32),
            pltpu.VMEM((PER_W,), jnp.int32),
            pltpu.VMEM((PER_W,), jnp.int32),
            pltpu.VMEM((PER_W,), jnp.int32),
            pltpu.VMEM((PER_W,), jnp.int32),
            pltpu.VMEM((PER_W,), jnp.int32),
            pltpu.VMEM((CHUNK, EMBED_DIM), jnp.float32),
            pltpu.VMEM((CHUNK, EMBED_DIM), jnp.float32),
            pltpu.VMEM((CHUNK, EMBED_DIM), jnp.float32),
            pltpu.VMEM((CHUNK, EMBED_DIM), jnp.float32),
            pltpu.VMEM((CHUNK, EMBED_DIM), jnp.float32),
            pltpu.VMEM((CHUNK, EMBED_DIM), jnp.float32),
            pltpu.VMEM((CHUNK, EMBED_DIM), jnp.float32),
            pltpu.VMEM((CHUNK, EMBED_DIM), jnp.float32),
            pltpu.VMEM((RELATION_NUM, EMBED_DIM), jnp.float32),
            pltpu.VMEM((PER_W,), jnp.float32),
            pltpu.SemaphoreType.DMA,
            pltpu.SemaphoreType.DMA,
        ],
    )
    return f(h, r, t, tt, c_h, c_r, c_t, c_tt, a_rows, wb_rows, rel_emb)


def kernel(h, r, t, tt, c_h, c_r, c_t, c_tt,
           a_embedding, w_embedding, b_embedding, relation_embedding):
    i32 = jnp.int32
    a_pack, wb_pack = _repack(jnp.transpose(a_embedding),
                              jnp.transpose(w_embedding),
                              jnp.transpose(b_embedding))
    a_rows = a_pack.reshape(2 * ENTITY_NUM, EMBED_DIM)
    wb_rows = wb_pack.reshape(2 * ENTITY_NUM, EMBED_DIM)
    out = _score(h.astype(i32), r.astype(i32), t.astype(i32), tt.astype(i32),
                 c_h.astype(i32), c_r.astype(i32), c_t.astype(i32),
                 c_tt.astype(i32), a_rows, wb_rows, relation_embedding)
    return (out[0], out[1])


# speedup vs baseline: 3.3397x; 1.7529x over previous
import functools
import math

import jax
import jax.numpy as jnp
from jax import lax
from jax.experimental import pallas as pl
from jax.experimental.pallas import tpu as pltpu
from jax.experimental.pallas import tpu_sc as plsc

ENTITY_NUM = 100000
HALF_N = ENTITY_NUM // 2
RELATION_NUM = 500
EMBED_DIM = 64
DMOD = 19
BOFF = 24
BATCH = 16384
NUM_CORES = 2
NUM_SUBCORES = 16
NUM_WORKERS = NUM_CORES * NUM_SUBCORES
PER_W = BATCH // NUM_WORKERS
CHUNK = 128
NCHUNK = PER_W // CHUNK
GROUPS = CHUNK // 16
ROWW = 128
TB = 2048
NTB = (ENTITY_NUM + TB - 1) // TB

_INV_PI = float(1.0 / math.pi)
_PI_HI = 3.140625
_PI_LO = float(math.pi - 3.140625)
_MAGIC = 12582912.0
_S3 = -1.6666654611e-1
_S5 = 8.3321608736e-3
_S7 = -1.9515295891e-4


def _sin16(x):
    kf = x * _INV_PI
    kf = (kf + _MAGIC) - _MAGIC
    r = x - kf * _PI_HI
    r = r - kf * _PI_LO
    k = kf.astype(jnp.int32)
    r2 = r * r
    p = _S5 + r2 * _S7
    p = _S3 + r2 * p
    s = r + r * (r2 * p)
    return jnp.where((k & 1) == 1, -s, s)


def _repack_body(a_ref, w_ref, b_ref, ao_ref, wbo_ref):
    ao_ref[...] = jnp.concatenate(
        [a_ref[...].T, jnp.zeros((TB, ROWW - EMBED_DIM), jnp.float32)],
        axis=1)
    wt = w_ref[...].T
    bt = b_ref[...].T
    z13 = jnp.zeros((TB, 13), jnp.float32)
    z64 = jnp.zeros((TB, ROWW - EMBED_DIM), jnp.float32)
    wbo_ref[...] = jnp.concatenate(
        [wt[:, :16], wt[:, 16:], z13, bt[:, :16], bt[:, 16:], z13, z64],
        axis=1)


def _repack(at, wt, bt):
    return pl.pallas_call(
        _repack_body,
        grid=(NTB,),
        in_specs=[
            pl.BlockSpec((EMBED_DIM, TB), lambda i: (0, i)),
            pl.BlockSpec((DMOD, TB), lambda i: (0, i)),
            pl.BlockSpec((DMOD, TB), lambda i: (0, i)),
        ],
        out_specs=[
            pl.BlockSpec((TB, ROWW), lambda i: (i, 0)),
            pl.BlockSpec((TB, ROWW), lambda i: (i, 0)),
        ],
        out_shape=[
            jax.ShapeDtypeStruct((ENTITY_NUM, ROWW), jnp.float32),
            jax.ShapeDtypeStruct((ENTITY_NUM, ROWW), jnp.float32),
        ],
    )(at, wt, bt)


def _body(h_hbm, r_hbm, t_hbm, tt_hbm, c_h_hbm, c_r_hbm, c_t_hbm, c_tt_hbm,
          a_hbm, wb_hbm, rel_hbm, correct_hbm, corrupt_hbm,
          hi_v, ti_v, chi_v, cti_v,
          a1_v, wb1_v, a2_v, wb2_v, a1b_v, wb1b_v, a2b_v, wb2b_v,
          rel_v, out_v, ri_v, tti_v, cri_v, ctti_v, sem_a, sem_b):
    wid = lax.axis_index("s") * NUM_CORES + lax.axis_index("c")
    wbase = wid * PER_W

    pltpu.sync_copy(h_hbm.at[pl.ds(wbase, PER_W)], hi_v)
    pltpu.sync_copy(t_hbm.at[pl.ds(wbase, PER_W)], ti_v)
    pltpu.sync_copy(c_h_hbm.at[pl.ds(wbase, PER_W)], chi_v)
    pltpu.sync_copy(c_t_hbm.at[pl.ds(wbase, PER_W)], cti_v)
    pltpu.sync_copy(r_hbm.at[pl.ds(wbase, PER_W)], ri_v)
    pltpu.sync_copy(tt_hbm.at[pl.ds(wbase, PER_W)], tti_v)
    pltpu.sync_copy(c_r_hbm.at[pl.ds(wbase, PER_W)], cri_v)
    pltpu.sync_copy(c_tt_hbm.at[pl.ds(wbase, PER_W)], ctti_v)
    pltpu.sync_copy(rel_hbm, rel_v)

    @pl.loop(0, PER_W // 16)
    def _remap(i):
        s = pl.ds(i * 16, 16)
        for ref in (hi_v, ti_v, chi_v, cti_v):
            ref[s] = ref[s] * 2

    bufs = ((a1_v, wb1_v, a2_v, wb2_v, sem_a),
            (a1b_v, wb1b_v, a2b_v, wb2b_v, sem_b))

    def process(e1_v, e2_v, rr_v, time_v, out_hbm):
        def fire(ci, par):
            A1, WB1, A2, WB2, sem = bufs[par]
            cs = pl.ds(ci * CHUNK, CHUNK)
            pltpu.async_copy(a_hbm.at[e1_v.at[cs]], A1, sem)
            pltpu.async_copy(wb_hbm.at[e1_v.at[cs]], WB1, sem)
            pltpu.async_copy(a_hbm.at[e2_v.at[cs]], A2, sem)
            pltpu.async_copy(wb_hbm.at[e2_v.at[cs]], WB2, sem)

        def drain(ci, par):
            A1, WB1, A2, WB2, sem = bufs[par]
            cs = pl.ds(ci * CHUNK, CHUNK)
            pltpu.make_async_copy(a_hbm.at[e1_v.at[cs]], A1, sem).wait()
            pltpu.make_async_copy(wb_hbm.at[e1_v.at[cs]], WB1, sem).wait()
            pltpu.make_async_copy(a_hbm.at[e2_v.at[cs]], A2, sem).wait()
            pltpu.make_async_copy(wb_hbm.at[e2_v.at[cs]], WB2, sem).wait()

        lane = lax.iota(jnp.int32, 16)
        hi3 = lane < (DMOD - 16)

        def compute(ci, par):
            A1, WB1, A2, WB2, _ = bufs[par]

            @pl.loop(0, GROUPS)
            def _grp(g):
                base0 = g * 16
                gbase = ci * CHUNK + base0
                tt16 = time_v[pl.ds(gbase, 16)].astype(jnp.float32)
                r16 = rr_v[pl.ds(gbase, 16)]
                outacc = jnp.zeros((16,), jnp.float32)
                for e0 in range(16):
                    le = base0 + e0
                    ttf = jnp.full((16,), tt16[e0])
                    ri = r16[e0]
                    ones = jnp.ones((16,), jnp.float32)
                    s1lo = _sin16(WB1[le, pl.ds(0, 16)] * ttf
                                  + WB1[le, pl.ds(32, 16)])
                    s1hi = _sin16(WB1[le, pl.ds(16, 16)] * ttf
                                  + WB1[le, pl.ds(48, 16)])
                    s2lo = _sin16(WB2[le, pl.ds(0, 16)] * ttf
                                  + WB2[le, pl.ds(32, 16)])
                    s2hi = _sin16(WB2[le, pl.ds(16, 16)] * ttf
                                  + WB2[le, pl.ds(48, 16)])
                    m1 = jnp.where(hi3, s1hi, ones)
                    m2 = jnp.where(hi3, s2hi, ones)
                    d0 = (A1[le, pl.ds(0, 16)] * s1lo
                          + rel_v[ri, pl.ds(0, 16)]
                          - A2[le, pl.ds(0, 16)] * s2lo)
                    d1 = (A1[le, pl.ds(16, 16)] * m1
                          + rel_v[ri, pl.ds(16, 16)]
                          - A2[le, pl.ds(16, 16)] * m2)
                    d2 = (A1[le, pl.ds(32, 16)]
                          + rel_v[ri, pl.ds(32, 16)]
                          - A2[le, pl.ds(32, 16)])
                    d3 = (A1[le, pl.ds(48, 16)]
                          + rel_v[ri, pl.ds(48, 16)]
                          - A2[le, pl.ds(48, 16)])
                    q = d0 * d0 + d1 * d1 + d2 * d2 + d3 * d3
                    outacc = jnp.where(lane == e0, jnp.sum(q), outacc)
                out_v[pl.ds(gbase, 16)] = outacc

        fire(0, 0)

        @pl.loop(0, NCHUNK // 2)
        def _chunk2(cj):
            ci0 = cj * 2
            fire(ci0 + 1, 1)
            drain(ci0, 0)
            compute(ci0, 0)

            @pl.when(ci0 + 2 < NCHUNK)
            def _prefetch():
                fire(ci0 + 2, 0)

            drain(ci0 + 1, 1)
            compute(ci0 + 1, 1)

        pltpu.sync_copy(out_v, out_hbm.at[pl.ds(wbase, PER_W)])

    process(hi_v, ti_v, ri_v, tti_v, correct_hbm)
    process(chi_v, cti_v, cri_v, ctti_v, corrupt_hbm)


def _score(h, r, t, tt, c_h, c_r, c_t, c_tt, a_rows, wb_rows, rel_emb):
    f = pl.kernel(
        _body,
        out_type=(
            jax.ShapeDtypeStruct((BATCH,), jnp.float32),
            jax.ShapeDtypeStruct((BATCH,), jnp.float32),
        ),
        compiler_params=pltpu.CompilerParams(
            needs_layout_passes=False, use_tc_tiling_on_sc=False),
        mesh=plsc.VectorSubcoreMesh(
            core_axis_name="c", subcore_axis_name="s",
            num_cores=NUM_CORES, num_subcores=NUM_SUBCORES),
        scratch_types=[
            pltpu.VMEM((PER_W,), jnp.int32),
            pltpu.VMEM((PER_W,), jnp.int32),
            pltpu.VMEM((PER_W,), jnp.int32),
            pltpu.VMEM((PER_W,), jnp.int32),
            pltpu.VMEM((CHUNK, EMBED_DIM), jnp.float32),
            pltpu.VMEM((CHUNK, EMBED_DIM), jnp.float32),
            pltpu.VMEM((CHUNK, EMBED_DIM), jnp.float32),
            pltpu.VMEM((CHUNK, EMBED_DIM), jnp.float32),
            pltpu.VMEM((CHUNK, EMBED_DIM), jnp.float32),
            pltpu.VMEM((CHUNK, EMBED_DIM), jnp.float32),
            pltpu.VMEM((CHUNK, EMBED_DIM), jnp.float32),
            pltpu.VMEM((CHUNK, EMBED_DIM), jnp.float32),
            pltpu.VMEM((RELATION_NUM, EMBED_DIM), jnp.float32),
            pltpu.VMEM((PER_W,), jnp.float32),
            pltpu.VMEM((PER_W,), jnp.int32),
            pltpu.VMEM((PER_W,), jnp.int32),
            pltpu.VMEM((PER_W,), jnp.int32),
            pltpu.VMEM((PER_W,), jnp.int32),
            pltpu.SemaphoreType.DMA,
            pltpu.SemaphoreType.DMA,
        ],
    )
    return f(h, r, t, tt, c_h, c_r, c_t, c_tt, a_rows, wb_rows, rel_emb)


def kernel(h, r, t, tt, c_h, c_r, c_t, c_tt,
           a_embedding, w_embedding, b_embedding, relation_embedding):
    i32 = jnp.int32
    a_pack, wb_pack = _repack(jnp.transpose(a_embedding),
                              jnp.transpose(w_embedding),
                              jnp.transpose(b_embedding))
    a_rows = a_pack.reshape(2 * ENTITY_NUM, EMBED_DIM)
    wb_rows = wb_pack.reshape(2 * ENTITY_NUM, EMBED_DIM)
    out = _score(h.astype(i32), r.astype(i32), t.astype(i32), tt.astype(i32),
                 c_h.astype(i32), c_r.astype(i32), c_t.astype(i32),
                 c_tt.astype(i32), a_rows, wb_rows, relation_embedding)
    return (out[0], out[1])


# speedup vs baseline: 3.3494x; 1.0029x over previous
import functools
import math

import jax
import jax.numpy as jnp
from jax import lax
from jax.experimental import pallas as pl
from jax.experimental.pallas import tpu as pltpu
from jax.experimental.pallas import tpu_sc as plsc

ENTITY_NUM = 100000
HALF_N = ENTITY_NUM // 2
RELATION_NUM = 500
EMBED_DIM = 64
DMOD = 19
BOFF = 24
BATCH = 16384
NUM_CORES = 2
NUM_SUBCORES = 16
NUM_WORKERS = NUM_CORES * NUM_SUBCORES
PER_W = BATCH // NUM_WORKERS
CHUNK = 128
NCHUNK = PER_W // CHUNK
GROUPS = CHUNK // 16
ROWW = 128
TB = 2048
NTB = (ENTITY_NUM + TB - 1) // TB

_INV_PI = float(1.0 / math.pi)
_PI_HI = 3.140625
_PI_LO = float(math.pi - 3.140625)
_MAGIC = 12582912.0
_S3 = -1.6666654611e-1
_S5 = 8.3321608736e-3
_S7 = -1.9515295891e-4


def _sin16(x):
    kf = x * _INV_PI
    kf = (kf + _MAGIC) - _MAGIC
    r = x - kf * _PI_HI
    r = r - kf * _PI_LO
    k = kf.astype(jnp.int32)
    r2 = r * r
    p = _S5 + r2 * _S7
    p = _S3 + r2 * p
    s = r + r * (r2 * p)
    return jnp.where((k & 1) == 1, -s, s)


def _repack_body(a_ref, w_ref, b_ref, o_ref):
    wt = w_ref[...].T
    bt = b_ref[...].T
    z13 = jnp.zeros((TB, 13), jnp.float32)
    o_ref[...] = jnp.concatenate(
        [a_ref[...].T,
         wt[:, :16], wt[:, 16:], z13, bt[:, :16], bt[:, 16:], z13],
        axis=1)


def _repack(at, wt, bt):
    return pl.pallas_call(
        _repack_body,
        grid=(NTB,),
        in_specs=[
            pl.BlockSpec((EMBED_DIM, TB), lambda i: (0, i)),
            pl.BlockSpec((DMOD, TB), lambda i: (0, i)),
            pl.BlockSpec((DMOD, TB), lambda i: (0, i)),
        ],
        out_specs=pl.BlockSpec((TB, ROWW), lambda i: (i, 0)),
        out_shape=jax.ShapeDtypeStruct((ENTITY_NUM, ROWW), jnp.float32),
    )(at, wt, bt)


def _body(h_hbm, r_hbm, t_hbm, tt_hbm, c_h_hbm, c_r_hbm, c_t_hbm, c_tt_hbm,
          tab_hbm, rel_hbm, correct_hbm, corrupt_hbm,
          hi_v, ti_v, chi_v, cti_v,
          e1a_v, e2a_v, e1b_v, e2b_v,
          rel_v, out_v, ri_v, tti_v, cri_v, ctti_v, sem_a, sem_b):
    wid = lax.axis_index("s") * NUM_CORES + lax.axis_index("c")
    wbase = wid * PER_W

    pltpu.sync_copy(h_hbm.at[pl.ds(wbase, PER_W)], hi_v)
    pltpu.sync_copy(t_hbm.at[pl.ds(wbase, PER_W)], ti_v)
    pltpu.sync_copy(c_h_hbm.at[pl.ds(wbase, PER_W)], chi_v)
    pltpu.sync_copy(c_t_hbm.at[pl.ds(wbase, PER_W)], cti_v)
    pltpu.sync_copy(r_hbm.at[pl.ds(wbase, PER_W)], ri_v)
    pltpu.sync_copy(tt_hbm.at[pl.ds(wbase, PER_W)], tti_v)
    pltpu.sync_copy(c_r_hbm.at[pl.ds(wbase, PER_W)], cri_v)
    pltpu.sync_copy(c_tt_hbm.at[pl.ds(wbase, PER_W)], ctti_v)
    pltpu.sync_copy(rel_hbm, rel_v)

    bufs = ((e1a_v, e2a_v, sem_a), (e1b_v, e2b_v, sem_b))

    def process(e1_v, e2_v, rr_v, time_v, out_hbm):
        def fire(ci, par):
            T1, T2, sem = bufs[par]
            cs = pl.ds(ci * CHUNK, CHUNK)
            pltpu.async_copy(tab_hbm.at[e1_v.at[cs]], T1, sem)
            pltpu.async_copy(tab_hbm.at[e2_v.at[cs]], T2, sem)

        def drain(ci, par):
            T1, T2, sem = bufs[par]
            cs = pl.ds(ci * CHUNK, CHUNK)
            pltpu.make_async_copy(tab_hbm.at[e1_v.at[cs]], T1, sem).wait()
            pltpu.make_async_copy(tab_hbm.at[e2_v.at[cs]], T2, sem).wait()

        lane = lax.iota(jnp.int32, 16)
        hi3 = lane < (DMOD - 16)

        def compute(ci, par):
            T1, T2, _ = bufs[par]

            @pl.loop(0, GROUPS)
            def _grp(g):
                base0 = g * 16
                gbase = ci * CHUNK + base0
                tt16 = time_v[pl.ds(gbase, 16)].astype(jnp.float32)
                r16 = rr_v[pl.ds(gbase, 16)]
                outacc = jnp.zeros((16,), jnp.float32)
                for e0 in range(16):
                    le = base0 + e0
                    ttf = jnp.full((16,), tt16[e0])
                    ri = r16[e0]
                    ones = jnp.ones((16,), jnp.float32)
                    s1lo = _sin16(T1[le, pl.ds(64, 16)] * ttf
                                  + T1[le, pl.ds(96, 16)])
                    s1hi = _sin16(T1[le, pl.ds(80, 16)] * ttf
                                  + T1[le, pl.ds(112, 16)])
                    s2lo = _sin16(T2[le, pl.ds(64, 16)] * ttf
                                  + T2[le, pl.ds(96, 16)])
                    s2hi = _sin16(T2[le, pl.ds(80, 16)] * ttf
                                  + T2[le, pl.ds(112, 16)])
                    m1 = jnp.where(hi3, s1hi, ones)
                    m2 = jnp.where(hi3, s2hi, ones)
                    d0 = (T1[le, pl.ds(0, 16)] * s1lo
                          + rel_v[ri, pl.ds(0, 16)]
                          - T2[le, pl.ds(0, 16)] * s2lo)
                    d1 = (T1[le, pl.ds(16, 16)] * m1
                          + rel_v[ri, pl.ds(16, 16)]
                          - T2[le, pl.ds(16, 16)] * m2)
                    d2 = (T1[le, pl.ds(32, 16)]
                          + rel_v[ri, pl.ds(32, 16)]
                          - T2[le, pl.ds(32, 16)])
                    d3 = (T1[le, pl.ds(48, 16)]
                          + rel_v[ri, pl.ds(48, 16)]
                          - T2[le, pl.ds(48, 16)])
                    q = d0 * d0 + d1 * d1 + d2 * d2 + d3 * d3
                    outacc = jnp.where(lane == e0, jnp.sum(q), outacc)
                out_v[pl.ds(gbase, 16)] = outacc

        fire(0, 0)

        @pl.loop(0, NCHUNK // 2)
        def _chunk2(cj):
            ci0 = cj * 2
            fire(ci0 + 1, 1)
            drain(ci0, 0)
            compute(ci0, 0)

            @pl.when(ci0 + 2 < NCHUNK)
            def _prefetch():
                fire(ci0 + 2, 0)

            drain(ci0 + 1, 1)
            compute(ci0 + 1, 1)

        pltpu.sync_copy(out_v, out_hbm.at[pl.ds(wbase, PER_W)])

    process(hi_v, ti_v, ri_v, tti_v, correct_hbm)
    process(chi_v, cti_v, cri_v, ctti_v, corrupt_hbm)


def _score(h, r, t, tt, c_h, c_r, c_t, c_tt, tab_rows, rel_emb):
    f = pl.kernel(
        _body,
        out_type=(
            jax.ShapeDtypeStruct((BATCH,), jnp.float32),
            jax.ShapeDtypeStruct((BATCH,), jnp.float32),
        ),
        compiler_params=pltpu.CompilerParams(
            needs_layout_passes=False, use_tc_tiling_on_sc=False),
        mesh=plsc.VectorSubcoreMesh(
            core_axis_name="c", subcore_axis_name="s",
            num_cores=NUM_CORES, num_subcores=NUM_SUBCORES),
        scratch_types=[
            pltpu.VMEM((PER_W,), jnp.int32),
            pltpu.VMEM((PER_W,), jnp.int32),
            pltpu.VMEM((PER_W,), jnp.int32),
            pltpu.VMEM((PER_W,), jnp.int32),
            pltpu.VMEM((CHUNK, ROWW), jnp.float32),
            pltpu.VMEM((CHUNK, ROWW), jnp.float32),
            pltpu.VMEM((CHUNK, ROWW), jnp.float32),
            pltpu.VMEM((CHUNK, ROWW), jnp.float32),
            pltpu.VMEM((RELATION_NUM, EMBED_DIM), jnp.float32),
            pltpu.VMEM((PER_W,), jnp.float32),
            pltpu.VMEM((PER_W,), jnp.int32),
            pltpu.VMEM((PER_W,), jnp.int32),
            pltpu.VMEM((PER_W,), jnp.int32),
            pltpu.VMEM((PER_W,), jnp.int32),
            pltpu.SemaphoreType.DMA,
            pltpu.SemaphoreType.DMA,
        ],
    )
    return f(h, r, t, tt, c_h, c_r, c_t, c_tt, tab_rows, rel_emb)


def kernel(h, r, t, tt, c_h, c_r, c_t, c_tt,
           a_embedding, w_embedding, b_embedding, relation_embedding):
    i32 = jnp.int32
    tab_rows = _repack(jnp.transpose(a_embedding),
                       jnp.transpose(w_embedding),
                       jnp.transpose(b_embedding))
    out = _score(h.astype(i32), r.astype(i32), t.astype(i32), tt.astype(i32),
                 c_h.astype(i32), c_r.astype(i32), c_t.astype(i32),
                 c_tt.astype(i32), tab_rows, relation_embedding)
    return (out[0], out[1])
